# baseline (speedup 1.0000x reference)
import dataclasses
import functools

import jax
import jax.numpy as jnp
from jax import lax
from jax.experimental import pallas as pl
from jax.experimental.pallas import tpu as pltpu
from jax.experimental.pallas import tpu_sc as plsc

NLAT_IN, NLON_IN = 46, 90
NLAT_OUT, NLON_OUT = 92, 180
K = 9
C_IN = 128
C_OUT = 64
NNZ = 30000
NROWS = K * NLAT_IN
QROWS = 23
NOUT_ROWS = NLAT_OUT * 2 * QROWS
NOUT_PAD = 4352
CH = 32
NSUB = 16
NCORE = 2
NCHKS = 2
CSZ = 148
NZ_PER_SUB = NCHKS * CSZ
NSC = NSUB * NZ_PER_SUB
NTC_CHUNKS = 16
TC_CHUNK = 1580
NTC = NTC_CHUNKS * TC_CHUNK
NPAD = NSC + NTC
ROWS_PER_SUB = NOUT_PAD // NSUB
OCHUNK = 16
SROWS = 24
LW = 2 * NLON_IN - 2 * 16
MROW = 4 * CSZ
GROUP = 4


def _mix_body(x_ref, w_ref, out_ref):
    xkk = jax.lax.dot_general(
        x_ref[...], w_ref[0, 0],
        (((0,), (0,)), ((), ())),
        preferred_element_type=jnp.float32,
    )
    xkk = xkk.reshape(NLAT_IN, NLON_IN, CH)
    out_ref[0, :, pl.ds(0, NLON_IN), :] = xkk
    out_ref[0, :, pl.ds(NLON_IN, NLON_IN), :] = xkk


def _channel_mix(x, weight):
    xf = x.reshape(C_IN, NLAT_IN * NLON_IN)
    wt = weight.transpose(2, 1, 0)
    wt = wt.reshape(K, C_IN, NCORE, CH).transpose(0, 2, 1, 3)
    return pl.pallas_call(
        _mix_body,
        grid=(K, NCORE),
        in_specs=[
            pl.BlockSpec((C_IN, NLAT_IN * NLON_IN), lambda k, h: (0, 0)),
            pl.BlockSpec((1, 1, C_IN, CH), lambda k, h: (k, h, 0, 0)),
        ],
        out_specs=pl.BlockSpec((1, NLAT_IN, 2 * NLON_IN, CH),
                               lambda k, h: (h, k, 0, 0)),
        out_shape=jax.ShapeDtypeStruct((NCORE, NROWS, 2 * NLON_IN, CH),
                                       jnp.float32),
    )(xf, wt)


def _sc_scatter_body(xk2_hbm, meta_hbm, patt_hbm,
                     out_hbm, rowbuf, stgs, idxs, patt,
                     msmem, obuf, acc, sems):
    cid = lax.axis_index("c")
    sid = lax.axis_index("s")
    row0 = pl.multiple_of(sid * ROWS_PER_SUB, 8)

    vz = jnp.zeros((16,), jnp.float32)

    @pl.loop(0, OCHUNK)
    def _zero(rr):
        for o in range(0, 128, 16):
            obuf[rr, pl.ds(o, 16)] = vz

    @pl.loop(0, ROWS_PER_SUB // OCHUNK)
    def _zacc(cc):
        pltpu.sync_copy(obuf, acc.at[pl.ds(row0 + cc * OCHUNK, OCHUNK)])

    pltpu.sync_copy(patt_hbm, patt)
    for stg in stgs:
        for g in range(SROWS):
            for o in range(0, 128, 16):
                stg[g, pl.ds(o, 16)] = vz
    for j, o in ((0, 0), (1, 8)):
        pj = patt[j]
        for idx in idxs:
            idx[pl.ds(o, 16)] = pj
    plsc.subcore_barrier()

    for stg, idx, sem in zip(stgs, idxs, sems):
        pltpu.async_copy(stg, acc.at[idx], sem, add=True)

    def do_one(stg, idx, sem, ktin, s, db, val, prev):
        pltpu.make_async_copy(stg, acc.at[idx], sem).wait()

        @pl.when(ktin != prev)
        def _():
            pltpu.sync_copy(xk2_hbm.at[cid, ktin], rowbuf)

        vval = jnp.full((16,), val, jnp.float32)
        for q in range(NLON_IN):
            r = s + q
            lo = 32 * (q % 4)
            stg[q // 4, pl.ds(lo, 16)] = rowbuf[r, pl.ds(0, 16)] * vval
            stg[q // 4, pl.ds(lo + 16, 16)] = rowbuf[r, pl.ds(16, 16)] * vval
        vdb = jnp.full((16,), db, jnp.int32)
        for j, o in ((0, 0), (1, 8)):
            idx[pl.ds(o, 16)] = patt[j] + vdb
        pltpu.async_copy(stg, acc.at[idx], sem, add=True)
        return ktin

    @pl.loop(0, NCHKS)
    def _chunk(cb):
        moff = pl.multiple_of((sid * NCHKS + cb) * MROW, 8)
        pltpu.sync_copy(meta_hbm.at[pl.ds(moff, MROW)], msmem)

        def group(i, prev):
            gidx = jnp.full((16,), 16 * i, jnp.int32) + lax.iota(jnp.int32, 16)
            mv = plsc.load_gather(msmem, [gidx])
            mvf = plsc.bitcast(mv, jnp.float32)
            for g in range(GROUP):
                prev = do_one(stgs[g], idxs[g], sems[g],
                              mv[4 * g], mv[4 * g + 1], mv[4 * g + 2],
                              mvf[4 * g + 3], prev)
            return prev

        lax.fori_loop(0, CSZ // GROUP, group, jnp.int32(-1))

    for stg, idx, sem in zip(stgs, idxs, sems):
        pltpu.make_async_copy(stg, acc.at[idx], sem).wait()
    plsc.subcore_barrier()
    @pl.loop(0, ROWS_PER_SUB // OCHUNK)
    def _out(cc):
        o0 = row0 + cc * OCHUNK
        pltpu.sync_copy(acc.at[pl.ds(o0, OCHUNK)], obuf)
        pltpu.sync_copy(obuf, out_hbm.at[cid, pl.ds(o0, OCHUNK)])


_SC_PARAMS = pltpu.CompilerParams()
if "needs_layout_passes" in pltpu.CompilerParams.__dataclass_fields__:
    _SC_PARAMS = dataclasses.replace(_SC_PARAMS, needs_layout_passes=False)


@functools.partial(
    pl.kernel,
    compiler_params=_SC_PARAMS,
    out_type=jax.ShapeDtypeStruct((NCORE, NOUT_PAD, 4 * CH), jnp.float32),
    mesh=plsc.VectorSubcoreMesh(core_axis_name="c", subcore_axis_name="s",
                                num_cores=NCORE, num_subcores=NSUB),
    scratch_types=[
        pltpu.VMEM((2 * NLON_IN, CH), jnp.float32),
        [pltpu.VMEM((SROWS, 4 * CH), jnp.float32)] * GROUP,
        [pltpu.VMEM((SROWS,), jnp.int32)] * GROUP,
        pltpu.VMEM((2, 16), jnp.int32),
        pltpu.VMEM((MROW,), jnp.int32),
        pltpu.VMEM((OCHUNK, 4 * CH), jnp.float32),
        pltpu.VMEM_SHARED((NOUT_PAD, 4 * CH), jnp.float32),
        [pltpu.SemaphoreType.DMA] * GROUP,
    ],
)
def _sc_scatter(xk2_hbm, meta_hbm, patt_hbm, out_hbm,
                rowbuf, stgs, idxs, patt, msmem, obuf, acc, sems):
    _sc_scatter_body(xk2_hbm, meta_hbm, patt_hbm,
                     out_hbm, rowbuf, stgs, idxs, patt,
                     msmem, obuf, acc, sems)


def _mix_body_tc(x_ref, w_ref, out_ref):
    xkk = jax.lax.dot_general(
        x_ref[...], w_ref[0],
        (((0,), (0,)), ((), ())),
        preferred_element_type=jnp.float32,
    )
    xkk = xkk.reshape(NLAT_IN, NLON_IN, C_OUT)
    out_ref[0, :, pl.ds(0, NLON_IN), :] = xkk
    out_ref[0, :, pl.ds(NLON_IN, NLON_IN), :] = xkk


def _tc_scatter_body(xk2_ref, ktin_ref, s_ref, db_ref, val_ref, out_ref, acc):
    g = pl.program_id(0)

    @pl.when(g == 0)
    def _zero():
        acc[...] = jnp.zeros_like(acc)

    def step(e, carry):
        ktin = ktin_ref[0, 0, e]
        s = s_ref[0, 0, e]
        db = db_ref[0, 0, e]
        v = val_ref[0, 0, e]
        src = xk2_ref[ktin, pl.ds(s, NLON_IN), :]
        cur = acc[pl.ds(db, NLON_IN), :]
        acc[pl.ds(db, NLON_IN), :] = cur + v * src
        return carry

    jax.lax.fori_loop(0, TC_CHUNK, step, 0)

    @pl.when(g == NTC_CHUNKS - 1)
    def _flush():
        out_ref[...] = acc[...]


def _tc_scatter(x, weight, ktin_c, s_c, db_c, val_c):
    xf = x.reshape(C_IN, NLAT_IN * NLON_IN)
    wt = weight.transpose(2, 1, 0)

    xk2_tc = pl.pallas_call(
        _mix_body_tc,
        grid=(K,),
        in_specs=[
            pl.BlockSpec((C_IN, NLAT_IN * NLON_IN), lambda k: (0, 0)),
            pl.BlockSpec((1, C_IN, C_OUT), lambda k: (k, 0, 0)),
        ],
        out_specs=pl.BlockSpec((1, NLAT_IN, 2 * NLON_IN, C_OUT),
                               lambda k: (k, 0, 0, 0)),
        out_shape=jax.ShapeDtypeStruct((K, NLAT_IN, 2 * NLON_IN, C_OUT),
                                       jnp.float32),
    )(xf, wt).reshape(NROWS, 2 * NLON_IN, C_OUT)

    return pl.pallas_call(
        _tc_scatter_body,
        grid=(NTC_CHUNKS,),
        in_specs=[
            pl.BlockSpec((NROWS, 2 * NLON_IN, C_OUT), lambda g: (0, 0, 0)),
            pl.BlockSpec((1, 1, TC_CHUNK), lambda g: (g, 0, 0),
                         memory_space=pltpu.SMEM),
            pl.BlockSpec((1, 1, TC_CHUNK), lambda g: (g, 0, 0),
                         memory_space=pltpu.SMEM),
            pl.BlockSpec((1, 1, TC_CHUNK), lambda g: (g, 0, 0),
                         memory_space=pltpu.SMEM),
            pl.BlockSpec((1, 1, TC_CHUNK), lambda g: (g, 0, 0),
                         memory_space=pltpu.SMEM),
        ],
        out_specs=pl.BlockSpec((NLAT_OUT * NLON_OUT, C_OUT), lambda g: (0, 0)),
        out_shape=jax.ShapeDtypeStruct((NLAT_OUT * NLON_OUT, C_OUT),
                                       jnp.float32),
        scratch_shapes=[
            pltpu.VMEM((NLAT_OUT * NLON_OUT, C_OUT), jnp.float32),
        ],
    )(xk2_tc, ktin_c, s_c, db_c, val_c)


def kernel(x, weight, bias, psi_ker_idx, psi_row_idx, psi_col_idx, psi_vals):
    ker = psi_ker_idx.astype(jnp.int32)
    tin = psi_row_idx.astype(jnp.int32)
    col = psi_col_idx.astype(jnp.int32)

    tout = col // NLON_OUT
    pout = (NLON_OUT - 1) - (col % NLON_OUT)
    m = pout // 2
    p = 1 - (pout % 2)
    s = (m + 1) % NLON_IN
    ktin = ker * NLAT_IN + tin
    db = (tout * 2 + p) * QROWS

    db_tc = (tout * 2 + p) * NLON_IN

    order = jnp.argsort(ktin)
    pad = NPAD - NNZ
    zi = jnp.zeros((pad,), jnp.int32)
    ktin_p = jnp.concatenate([ktin[order], zi])
    s_p = jnp.concatenate([s[order], zi])
    db_p = jnp.concatenate([db[order], zi])
    dbtc_p = jnp.concatenate([db_tc[order], zi])
    val_p = jnp.concatenate([psi_vals[order], jnp.zeros((pad,), jnp.float32)])

    valbits = lax.bitcast_convert_type(val_p[:NSC], jnp.int32)
    meta = jnp.stack([ktin_p[:NSC], s_p[:NSC], db_p[:NSC], valbits], axis=1)
    meta = meta.reshape(NSUB * NCHKS * MROW)

    patt = jnp.stack([jnp.arange(16, dtype=jnp.int32),
                      jnp.concatenate([jnp.arange(8, QROWS, dtype=jnp.int32),
                                       jnp.zeros((1,), jnp.int32)])])

    ktin_c = ktin_p[NSC:].reshape(NTC_CHUNKS, 1, TC_CHUNK)
    s_c = s_p[NSC:].reshape(NTC_CHUNKS, 1, TC_CHUNK)
    dbtc_c = dbtc_p[NSC:].reshape(NTC_CHUNKS, 1, TC_CHUNK)
    val_c = val_p[NSC:].reshape(NTC_CHUNKS, 1, TC_CHUNK)

    xk2 = _channel_mix(x, weight)
    acc = _sc_scatter(xk2, meta, patt)
    acc_tc = _tc_scatter(x, weight, ktin_c, s_c, dbtc_c, val_c)

    out = acc[:, :NOUT_ROWS].reshape(NCORE, NLAT_OUT, 2, QROWS, 4, CH)
    out = out.transpose(0, 5, 1, 3, 4, 2)
    out = out.reshape(NCORE, CH, NLAT_OUT, 4 * QROWS, 2)[:, :, :, :NLON_IN, :]
    out = out.reshape(1, C_OUT, NLAT_OUT, NLON_OUT)

    out_tc = acc_tc.reshape(NLAT_OUT, 2, NLON_IN, C_OUT)
    out_tc = out_tc.transpose(3, 0, 2, 1).reshape(1, C_OUT, NLAT_OUT, NLON_OUT)
    return out + out_tc + bias.reshape(1, -1, 1, 1)
